# Initial kernel scaffold; baseline (speedup 1.0000x reference)
import jax
import jax.numpy as jnp
from jax.experimental import pallas as pl


def kernel(x, edge_index, W0, b0, a0, W1, b1, a1, W2, b2, a2, Wh, bh):
    raise NotImplementedError("write your pallas kernel here")


# speedup vs baseline: 4.5682x; 4.5682x over previous
import jax
import jax.numpy as jnp
from jax import lax
from jax.experimental import pallas as pl
from jax.experimental.pallas import tpu as pltpu
from jax.experimental.pallas import tpu_sc as plsc

N = 10000
E = 320000
D = 128

NC = 2
NS = 16
NW = NC * NS
EPW = E // NW
K = 80
NB = EPW // K
NP = 10240
RPS = NP // NS
ZR = 128

BR = 1000
NBLK = N // BR


def _edge_body(p_hbm, q_hbm, dst_hbm, src_hbm, a_hbm, out_hbm,
               idx_d, idx_s, bufp, bufq, avec, zbuf, acc):
    c = lax.axis_index("c")
    s = lax.axis_index("s")
    wid = s * NC + c

    pltpu.sync_copy(a_hbm, avec)

    @pl.loop(0, ZR)
    def _(r):
        @pl.loop(0, D, step=16)
        def _(cc):
            zbuf[r, pl.ds(cc, 16)] = jnp.zeros((16,), jnp.float32)

    @pl.loop(0, RPS // ZR)
    def _(j):
        pltpu.sync_copy(zbuf, acc.at[pl.ds(s * RPS + j * ZR, ZR), :])

    plsc.subcore_barrier()

    base = wid * EPW

    @pl.loop(0, NB)
    def _(g):
        off = base + g * K
        pltpu.sync_copy(dst_hbm.at[pl.ds(off, K)], idx_d)
        pltpu.sync_copy(src_hbm.at[pl.ds(off, K)], idx_s)
        pltpu.sync_copy(p_hbm.at[idx_d], bufp)
        pltpu.sync_copy(q_hbm.at[idx_s], bufq)
        a = avec[...]

        @pl.loop(0, K)
        def _(r):
            @pl.loop(0, D, step=16)
            def _(cc):
                m = bufp[r, pl.ds(cc, 16)] + bufq[r, pl.ds(cc, 16)]
                bufp[r, pl.ds(cc, 16)] = (
                    jnp.maximum(m, 0.0) + a * jnp.minimum(m, 0.0))

        pltpu.sync_copy(bufp, acc.at[idx_d], add=True)

    plsc.subcore_barrier()

    pltpu.sync_copy(acc.at[pl.ds(s * RPS, RPS), :],
                    out_hbm.at[c, pl.ds(s * RPS, RPS), :])


_edge_call = pl.kernel(
    _edge_body,
    out_type=jax.ShapeDtypeStruct((NC, NP, D), jnp.float32),
    mesh=plsc.VectorSubcoreMesh(core_axis_name="c", subcore_axis_name="s"),
    scratch_types=[
        pltpu.VMEM((K,), jnp.int32),
        pltpu.VMEM((K,), jnp.int32),
        pltpu.VMEM((K, D), jnp.float32),
        pltpu.VMEM((K, D), jnp.float32),
        pltpu.VMEM((16,), jnp.float32),
        pltpu.VMEM((ZR, D), jnp.float32),
        pltpu.VMEM_SHARED((NP, D), jnp.float32),
    ],
)


def _dot(a, b):
    return lax.dot_general(a, b, (((1,), (0,)), ((), ())),
                           preferred_element_type=jnp.float32,
                           precision=lax.Precision.HIGHEST)


def _pq_first_body(x_ref, w_ref, b_ref, p_ref, q_ref):
    h = x_ref[...]
    p_ref[...] = _dot(h, w_ref[0:D, :]) + b_ref[...]
    q_ref[...] = _dot(h, w_ref[D:2 * D, :])


def _pq_next_body(parts_ref, w_ref, b_ref, p_ref, q_ref):
    h = jax.nn.sigmoid(parts_ref[0] + parts_ref[1])
    p_ref[...] = _dot(h, w_ref[0:D, :]) + b_ref[...]
    q_ref[...] = _dot(h, w_ref[D:2 * D, :])


def _head_body(parts_ref, wh_ref, bh_ref, o_ref):
    h = jax.nn.sigmoid(parts_ref[0] + parts_ref[1])
    z = jnp.sum(h * wh_ref[...], axis=1, keepdims=True) + bh_ref[...]
    o_ref[...] = jax.nn.sigmoid(z)


def _pq_first(x, W, b):
    return pl.pallas_call(
        _pq_first_body,
        grid=(NBLK,),
        in_specs=[pl.BlockSpec((BR, D), lambda i: (i, 0)),
                  pl.BlockSpec((2 * D, D), lambda i: (0, 0)),
                  pl.BlockSpec((1, D), lambda i: (0, 0))],
        out_specs=[pl.BlockSpec((BR, D), lambda i: (i, 0)),
                   pl.BlockSpec((BR, D), lambda i: (i, 0))],
        out_shape=[jax.ShapeDtypeStruct((N, D), jnp.float32)] * 2,
    )(x, W, b)


def _pq_next(parts, W, b):
    return pl.pallas_call(
        _pq_next_body,
        grid=(NBLK,),
        in_specs=[pl.BlockSpec((NC, BR, D), lambda i: (0, i, 0)),
                  pl.BlockSpec((2 * D, D), lambda i: (0, 0)),
                  pl.BlockSpec((1, D), lambda i: (0, 0))],
        out_specs=[pl.BlockSpec((BR, D), lambda i: (i, 0)),
                   pl.BlockSpec((BR, D), lambda i: (i, 0))],
        out_shape=[jax.ShapeDtypeStruct((N, D), jnp.float32)] * 2,
    )(parts, W, b)


def _head(parts, wh_row, bh):
    return pl.pallas_call(
        _head_body,
        grid=(NBLK,),
        in_specs=[pl.BlockSpec((NC, BR, D), lambda i: (0, i, 0)),
                  pl.BlockSpec((1, D), lambda i: (0, 0)),
                  pl.BlockSpec((1, 1), lambda i: (0, 0))],
        out_specs=pl.BlockSpec((BR, 1), lambda i: (i, 0)),
        out_shape=jax.ShapeDtypeStruct((N, 1), jnp.float32),
    )(parts, wh_row, bh)


def kernel(x, edge_index, W0, b0, a0, W1, b1, a1, W2, b2, a2, Wh, bh):
    a_vecs = [jnp.full((16,), a, jnp.float32) for a in (a0, a1, a2)]
    bs = [b.reshape(1, D) for b in (b0, b1, b2)]
    src = edge_index[0]
    dst = edge_index[1]

    P, Q = _pq_first(x, W0, bs[0])
    parts = _edge_call(P, Q, dst, src, a_vecs[0])
    P, Q = _pq_next(parts, W1, bs[1])
    parts = _edge_call(P, Q, dst, src, a_vecs[1])
    P, Q = _pq_next(parts, W2, bs[2])
    parts = _edge_call(P, Q, dst, src, a_vecs[2])
    out = _head(parts, Wh.reshape(1, D), bh.reshape(1, 1))
    return out.reshape(N)


# speedup vs baseline: 10.4047x; 2.2776x over previous
import jax
import jax.numpy as jnp
from jax import lax
from jax.experimental import pallas as pl
from jax.experimental.pallas import tpu as pltpu
from jax.experimental.pallas import tpu_sc as plsc

N = 10000
E = 320000
D = 128

NC = 2
NS = 16
NW = NC * NS
EPW = E // NW
K = 80
NB = EPW // K
NP = 10240
RPS = NP // NS
ZR = 128

BR = 1000
NBLK = N // BR


def _edge_body(p_hbm, q_hbm, dst_hbm, src_hbm, a_hbm, out_hbm,
               idx_d, idx_s, idx_w, bufp, bufq, avec, acc,
               sem_i, sem_g, sem_s):
    c = lax.axis_index("c")
    s = lax.axis_index("s")
    wid = s * NC + c

    pltpu.sync_copy(a_hbm, avec)

    @pl.loop(0, K)
    def _(r):
        for j in range(D // 16):
            bufp[0, r, pl.ds(j * 16, 16)] = jnp.zeros((16,), jnp.float32)

    @pl.loop(0, RPS // K)
    def _(j):
        pltpu.sync_copy(bufp.at[0], acc.at[pl.ds(s * RPS + j * K, K), :])

    plsc.subcore_barrier()

    base = wid * EPW

    def issue_idx(g, b):
        off = base + g * K
        pltpu.async_copy(dst_hbm.at[pl.ds(off, K)], idx_d.at[b], sem_i.at[b])
        pltpu.async_copy(src_hbm.at[pl.ds(off, K)], idx_s.at[b], sem_i.at[b])

    def wait_idx(b):
        pltpu.make_async_copy(dst_hbm.at[pl.ds(0, K)], idx_d.at[b],
                              sem_i.at[b]).wait()
        pltpu.make_async_copy(src_hbm.at[pl.ds(0, K)], idx_s.at[b],
                              sem_i.at[b]).wait()

    def issue_gathers(b):
        pltpu.async_copy(p_hbm.at[idx_d.at[b]], bufp.at[b], sem_g.at[b])
        pltpu.async_copy(q_hbm.at[idx_s.at[b]], bufq.at[b], sem_g.at[b])

    def wait_gathers(b):
        pltpu.make_async_copy(p_hbm.at[idx_d.at[b]], bufp.at[b],
                              sem_g.at[b]).wait()
        pltpu.make_async_copy(q_hbm.at[idx_s.at[b]], bufq.at[b],
                              sem_g.at[b]).wait()

    def issue_scatter(b):
        pltpu.async_copy(bufp.at[b], acc.at[idx_w.at[b]], sem_s.at[b],
                         add=True)

    def wait_scatter(b):
        pltpu.make_async_copy(bufp.at[b], acc.at[idx_w.at[b]],
                              sem_s.at[b]).wait()

    def compute(b):
        a = avec[...]

        @pl.loop(0, K)
        def _(r):
            for j in range(D // 16):
                sl = pl.ds(j * 16, 16)
                m = bufp[b, r, sl] + bufq[b, r, sl]
                bufp[b, r, sl] = jnp.maximum(m, 0.0) + a * jnp.minimum(m, 0.0)

    def copy_scatter_idx(b):
        for j in range(K // 16):
            sl = pl.ds(j * 16, 16)
            idx_w[b, sl] = idx_d[b, sl]

    def step(g, b):
        wait_gathers(b)
        copy_scatter_idx(b)

        @pl.when(g + 1 < NB)
        def _():
            wait_idx(1 - b)

        @pl.when((g >= 1) & (g + 1 < NB))
        def _():
            wait_scatter(1 - b)

        @pl.when(g + 1 < NB)
        def _():
            issue_gathers(1 - b)

        @pl.when(g + 2 < NB)
        def _():
            issue_idx(g + 2, b)

        compute(b)
        issue_scatter(b)

    issue_idx(0, 0)
    issue_idx(1, 1)
    wait_idx(0)
    issue_gathers(0)

    @pl.loop(0, NB // 2)
    def _(i):
        step(i * 2, 0)
        step(i * 2 + 1, 1)

    if NB % 2 == 1:
        wait_gathers(0)
        copy_scatter_idx(0)
        compute(0)
        issue_scatter(0)
        wait_scatter(1)
        wait_scatter(0)
    else:
        wait_scatter(0)
        wait_scatter(1)

    plsc.subcore_barrier()

    pltpu.sync_copy(acc.at[pl.ds(s * RPS, RPS), :],
                    out_hbm.at[c, pl.ds(s * RPS, RPS), :])


_edge_call = pl.kernel(
    _edge_body,
    out_type=jax.ShapeDtypeStruct((NC, NP, D), jnp.float32),
    mesh=plsc.VectorSubcoreMesh(core_axis_name="c", subcore_axis_name="s"),
    scratch_types=[
        pltpu.VMEM((2, K), jnp.int32),
        pltpu.VMEM((2, K), jnp.int32),
        pltpu.VMEM((2, K), jnp.int32),
        pltpu.VMEM((2, K, D), jnp.float32),
        pltpu.VMEM((2, K, D), jnp.float32),
        pltpu.VMEM((16,), jnp.float32),
        pltpu.VMEM_SHARED((NP, D), jnp.float32),
        pltpu.SemaphoreType.DMA((2,)),
        pltpu.SemaphoreType.DMA((2,)),
        pltpu.SemaphoreType.DMA((2,)),
    ],
)


def _dot(a, b):
    return lax.dot_general(a, b, (((1,), (0,)), ((), ())),
                           preferred_element_type=jnp.float32,
                           precision=lax.Precision.HIGHEST)


def _pq_first_body(x_ref, w_ref, b_ref, p_ref, q_ref):
    h = x_ref[...]
    p_ref[...] = _dot(h, w_ref[0:D, :]) + b_ref[...]
    q_ref[...] = _dot(h, w_ref[D:2 * D, :])


def _pq_next_body(parts_ref, w_ref, b_ref, p_ref, q_ref):
    h = jax.nn.sigmoid(parts_ref[0] + parts_ref[1])
    p_ref[...] = _dot(h, w_ref[0:D, :]) + b_ref[...]
    q_ref[...] = _dot(h, w_ref[D:2 * D, :])


def _head_body(parts_ref, wh_ref, bh_ref, o_ref):
    h = jax.nn.sigmoid(parts_ref[0] + parts_ref[1])
    z = jnp.sum(h * wh_ref[...], axis=1, keepdims=True) + bh_ref[...]
    o_ref[...] = jax.nn.sigmoid(z)


def _pq_first(x, W, b):
    return pl.pallas_call(
        _pq_first_body,
        grid=(NBLK,),
        in_specs=[pl.BlockSpec((BR, D), lambda i: (i, 0)),
                  pl.BlockSpec((2 * D, D), lambda i: (0, 0)),
                  pl.BlockSpec((1, D), lambda i: (0, 0))],
        out_specs=[pl.BlockSpec((BR, D), lambda i: (i, 0)),
                   pl.BlockSpec((BR, D), lambda i: (i, 0))],
        out_shape=[jax.ShapeDtypeStruct((N, D), jnp.float32)] * 2,
    )(x, W, b)


def _pq_next(parts, W, b):
    return pl.pallas_call(
        _pq_next_body,
        grid=(NBLK,),
        in_specs=[pl.BlockSpec((NC, BR, D), lambda i: (0, i, 0)),
                  pl.BlockSpec((2 * D, D), lambda i: (0, 0)),
                  pl.BlockSpec((1, D), lambda i: (0, 0))],
        out_specs=[pl.BlockSpec((BR, D), lambda i: (i, 0)),
                   pl.BlockSpec((BR, D), lambda i: (i, 0))],
        out_shape=[jax.ShapeDtypeStruct((N, D), jnp.float32)] * 2,
    )(parts, W, b)


def _head(parts, wh_row, bh):
    return pl.pallas_call(
        _head_body,
        grid=(NBLK,),
        in_specs=[pl.BlockSpec((NC, BR, D), lambda i: (0, i, 0)),
                  pl.BlockSpec((1, D), lambda i: (0, 0)),
                  pl.BlockSpec((1, 1), lambda i: (0, 0))],
        out_specs=pl.BlockSpec((BR, 1), lambda i: (i, 0)),
        out_shape=jax.ShapeDtypeStruct((N, 1), jnp.float32),
    )(parts, wh_row, bh)


def kernel(x, edge_index, W0, b0, a0, W1, b1, a1, W2, b2, a2, Wh, bh):
    a_vecs = [jnp.full((16,), a, jnp.float32) for a in (a0, a1, a2)]
    bs = [b.reshape(1, D) for b in (b0, b1, b2)]
    src = edge_index[0]
    dst = edge_index[1]

    P, Q = _pq_first(x, W0, bs[0])
    parts = _edge_call(P, Q, dst, src, a_vecs[0])
    P, Q = _pq_next(parts, W1, bs[1])
    parts = _edge_call(P, Q, dst, src, a_vecs[1])
    P, Q = _pq_next(parts, W2, bs[2])
    parts = _edge_call(P, Q, dst, src, a_vecs[2])
    out = _head(parts, Wh.reshape(1, D), bh.reshape(1, 1))
    return out.reshape(N)
